# Initial kernel scaffold; baseline (speedup 1.0000x reference)
import jax
import jax.numpy as jnp
from jax.experimental import pallas as pl


def kernel(h, pos, edge_index, params):
    raise NotImplementedError("write your pallas kernel here")


# speedup vs baseline: 1.2146x; 1.2146x over previous
import functools

import jax
import jax.numpy as jnp
from jax import lax
from jax.experimental import pallas as pl
from jax.experimental.pallas import tpu as pltpu
from jax.experimental.pallas import tpu_sc as plsc

N = 10000
E = 160000
N_PAD = 10240
N_SP = 10112
TRASH = 10000
E_PAD = 163840
BM = 1024
PREC = jax.lax.Precision.HIGHEST


def _stage_a_body(x_ref, pos_ref, wax_ref, wap_ref, ba_ref, g_ref, p_ref):
    p = jnp.dot(pos_ref[...], wap_ref[...], precision=PREC,
                preferred_element_type=jnp.float32)
    g = jnp.dot(x_ref[...], wax_ref[...], precision=PREC,
                preferred_element_type=jnp.float32)
    p_ref[0] = p
    g_ref[0] = g + p + ba_ref[...]


def _stage_a(x, pos, wax, wap, ba):
    fi, h = wax.shape
    nc = h // 128
    nm = N_PAD // BM
    out_shape = jax.ShapeDtypeStruct((nc, N_PAD, 128), jnp.float32)
    grid = (nm, nc)
    return pl.pallas_call(
        _stage_a_body,
        grid=grid,
        in_specs=[
            pl.BlockSpec((BM, fi), lambda m, o: (m, 0)),
            pl.BlockSpec((BM, 2), lambda m, o: (m, 0)),
            pl.BlockSpec((fi, 128), lambda m, o: (0, o)),
            pl.BlockSpec((2, 128), lambda m, o: (0, o)),
            pl.BlockSpec((1, 128), lambda m, o: (0, o)),
        ],
        out_specs=[
            pl.BlockSpec((1, BM, 128), lambda m, o: (o, m, 0)),
            pl.BlockSpec((1, BM, 128), lambda m, o: (o, m, 0)),
        ],
        out_shape=[out_shape, out_shape],
    )(x, pos, wax, wap, ba)


def _stage_c_body(nk, relu, s_ref, wb_ref, cnt_ref, bb_ref, o_ref, acc_ref):
    k = pl.program_id(2)

    @pl.when(k == 0)
    def _():
        acc_ref[...] = jnp.zeros_like(acc_ref)

    acc_ref[...] += jnp.dot(s_ref[0], wb_ref[0], precision=PREC,
                            preferred_element_type=jnp.float32)

    @pl.when(k == nk - 1)
    def _():
        csum = jnp.sum(cnt_ref[...], axis=(0, 2))[:, None]
        rc = 1.0 / jnp.maximum(csum, 1.0)
        ind = jnp.minimum(csum, 1.0)
        out = acc_ref[...] * rc + ind * bb_ref[...]
        if relu:
            out = jnp.maximum(out, 0.0)
        o_ref[...] = out


def _stage_c(s, wb3, cnt2, bb, relu):
    nk = wb3.shape[0]
    o = wb3.shape[2]
    bo = min(o, 256)
    nm = N_PAD // BM
    no = o // bo
    grid = (nm, no, nk)
    return pl.pallas_call(
        functools.partial(_stage_c_body, nk, relu),
        grid=grid,
        in_specs=[
            pl.BlockSpec((1, BM, 128), lambda m, o_, k: (k, m, 0)),
            pl.BlockSpec((1, 128, bo), lambda m, o_, k: (k, 0, o_)),
            pl.BlockSpec((2, BM, 16), lambda m, o_, k: (0, m, 0)),
            pl.BlockSpec((1, bo), lambda m, o_, k: (0, o_)),
        ],
        out_specs=pl.BlockSpec((BM, bo), lambda m, o_, k: (m, o_)),
        out_shape=jax.ShapeDtypeStruct((N_PAD, o), jnp.float32),
        scratch_shapes=[pltpu.VMEM((BM, bo), jnp.float32)],
    )(s, wb3, cnt2, bb)


def _dense_body(relu, x_ref, w_ref, b_ref, o_ref):
    out = jnp.dot(x_ref[...], w_ref[...], precision=PREC,
                  preferred_element_type=jnp.float32) + b_ref[...]
    if relu:
        out = jnp.maximum(out, 0.0)
    o_ref[...] = out


def _dense(x, w, b, relu):
    k, o = w.shape
    bo = min(o, 512)
    grid = (N_PAD // BM, o // bo)
    return pl.pallas_call(
        functools.partial(_dense_body, relu),
        grid=grid,
        in_specs=[
            pl.BlockSpec((BM, k), lambda m, o_: (m, 0)),
            pl.BlockSpec((k, bo), lambda m, o_: (0, o_)),
            pl.BlockSpec((1, bo), lambda m, o_: (0, o_)),
        ],
        out_specs=pl.BlockSpec((BM, bo), lambda m, o_: (m, o_)),
        out_shape=jax.ShapeDtypeStruct((N_PAD, o), jnp.float32),
    )(x, w, b)


_SC_MESH = plsc.VectorSubcoreMesh(core_axis_name="c", subcore_axis_name="s")
STRIPE = N_SP // 16


def _make_sc_edge(nc):
    edge_split = nc == 1
    passes = 1 if nc <= 2 else nc // 2
    n_out = 2 if nc == 1 else nc
    nb = 40 if edge_split else 80

    out_type = [jax.ShapeDtypeStruct((n_out * N_PAD, 128), jnp.float32)]

    scratch_types = [
        pltpu.VMEM((128,), jnp.int32),
        pltpu.VMEM((128,), jnp.int32),
        pltpu.VMEM((128,), jnp.int32),
        pltpu.VMEM((128,), jnp.int32),
        pltpu.VMEM((128, 128), jnp.float32),
        pltpu.VMEM((128, 128), jnp.float32),
        pltpu.VMEM_SHARED((N_SP, 128), jnp.float32),
        pltpu.SemaphoreType.DMA,
        pltpu.SemaphoreType.DMA,
    ]

    def body(g2, p2, src_hbm, dst_hbm, z128, s_out,
             idx_s, idx_d, idx_s2, idx_d2, rows_g, rows_p, s_sh, sem1, sem2):
        core = lax.axis_index("c")
        sid = lax.axis_index("s")
        r0 = sid * STRIPE

        if edge_split:
            ebase = core * (E_PAD // 2) + sid * (nb * 128)
        else:
            ebase = sid * (nb * 128)

        for p in range(passes):
            gc = 0 if edge_split else core * passes + p
            goff = jnp.full((16,), gc * N_PAD, jnp.int32)
            pltpu.sync_copy(z128, s_sh.at[pl.ds(r0, STRIPE)])
            plsc.subcore_barrier()

            @pl.loop(0, nb)
            def _(b):
                off = ebase + b * 128
                pltpu.sync_copy(src_hbm.at[pl.ds(off, 128)], idx_s)
                pltpu.sync_copy(dst_hbm.at[pl.ds(off, 128)], idx_d)
                for j in range(8):
                    sl = pl.ds(j * 16, 16)
                    idx_s2[sl] = idx_s[sl] + goff
                    idx_d2[sl] = idx_d[sl] + goff
                cg = pltpu.async_copy(g2.at[idx_s2], rows_g, sem1)
                cp = pltpu.async_copy(p2.at[idx_d2], rows_p, sem2)
                cg.wait()
                cp.wait()

                @pl.loop(0, 128)
                def _(r):
                    for c in range(8):
                        sl = pl.ds(c * 16, 16)
                        gv = rows_g[r, sl]
                        pv = rows_p[r, sl]
                        rows_g[r, sl] = jnp.maximum(gv - pv, 0.0)

                pltpu.sync_copy(rows_g, s_sh.at[idx_d], add=True)

            plsc.subcore_barrier()
            out_row = (core if nc <= 2 else gc) * N_PAD + r0
            pltpu.sync_copy(s_sh.at[pl.ds(r0, STRIPE)],
                            s_out.at[pl.ds(out_row, STRIPE)])

    return pl.kernel(body, mesh=_SC_MESH, out_type=out_type,
                     scratch_types=scratch_types)


def _sc_edge(g3, p3, src_p, dst_p, z128):
    nc = g3.shape[0]
    fn = _make_sc_edge(nc)
    g2 = g3.reshape(nc * N_PAD, 128)
    p2 = p3.reshape(nc * N_PAD, 128)
    out = fn(g2, p2, src_p, dst_p, z128)
    return out[0].reshape(-1, N_PAD, 128)


def _make_sc_cnt():
    nb = E_PAD // 32 // 128

    out_type = [jax.ShapeDtypeStruct((2 * N_PAD, 16), jnp.float32)]
    scratch_types = [
        pltpu.VMEM((128,), jnp.int32),
        pltpu.VMEM((128, 16), jnp.float32),
        pltpu.VMEM_SHARED((N_SP, 16), jnp.float32),
    ]

    def body(dst_hbm, z16, one16, cnt_out, idx_d, ones_v, cnt_sh):
        core = lax.axis_index("c")
        sid = lax.axis_index("s")
        r0 = sid * STRIPE
        pltpu.sync_copy(one16, ones_v)
        pltpu.sync_copy(z16, cnt_sh.at[pl.ds(r0, STRIPE)])
        plsc.subcore_barrier()
        ebase = (core * 16 + sid) * (nb * 128)

        @pl.loop(0, nb)
        def _(b):
            off = ebase + b * 128
            pltpu.sync_copy(dst_hbm.at[pl.ds(off, 128)], idx_d)
            pltpu.sync_copy(ones_v, cnt_sh.at[idx_d], add=True)

        plsc.subcore_barrier()
        cnt_row = core * N_PAD + r0
        pltpu.sync_copy(cnt_sh.at[pl.ds(r0, STRIPE)],
                        cnt_out.at[pl.ds(cnt_row, STRIPE)])

    return pl.kernel(body, mesh=_SC_MESH, out_type=out_type,
                     scratch_types=scratch_types)


def _sc_cnt(dst_p, z16, one16):
    out = _make_sc_cnt()(dst_p, z16, one16)
    return out[0].reshape(2, N_PAD, 16)


def kernel(h, pos, edge_index, params):
    p = params
    src = edge_index[0]
    dst = edge_index[1]
    src_p = jnp.pad(src, (0, E_PAD - E))
    dst_p = jnp.pad(dst, (0, E_PAD - E), constant_values=TRASH)

    x = jnp.pad(h, ((0, N_PAD - N), (0, 0)))
    pos_p = jnp.pad(pos, ((0, N_PAD - N), (0, 0)))

    z128 = jnp.zeros((N_SP // 16, 128), jnp.float32)
    z16 = jnp.zeros((N_SP // 16, 16), jnp.float32)
    one16 = jnp.ones((128, 16), jnp.float32)

    cnt2 = _sc_cnt(dst_p, z16, one16)
    for i in range(4):
        wa = p["W%d" % (2 * i)]
        ba = p["b%d" % (2 * i)]
        wb = p["W%d" % (2 * i + 1)]
        bb = p["b%d" % (2 * i + 1)]
        fi = wa.shape[0] - 2
        hdim = wa.shape[1]
        wax = wa[:fi]
        wap = wa[fi:]
        g3, p3 = _stage_a(x, pos_p, wax, wap, ba.reshape(1, -1))
        s3 = _sc_edge(g3, p3, src_p, dst_p, z128)
        if hdim == 128:
            wb3 = jnp.broadcast_to(wb[None], (2,) + wb.shape)
        else:
            wb3 = wb.reshape(hdim // 128, 128, wb.shape[1])
        x = _stage_c(s3, wb3, cnt2, bb.reshape(1, -1), True)

    x = _dense(x, p["W8"], p["b8"].reshape(1, -1), True)
    x = _dense(x, p["W9"], p["b9"].reshape(1, -1), True)
    w10 = jnp.pad(p["W10"], ((0, 0), (0, 128 - p["W10"].shape[1])))
    b10 = jnp.pad(p["b10"], (0, 128 - p["b10"].shape[0]))
    x = _dense(x, w10, b10.reshape(1, -1), False)
    return x[:N, :40]
